# baseline (speedup 1.0000x reference)
import jax
import jax.numpy as jnp
from jax.experimental import pallas as pl
from jax.experimental.pallas import tpu as pltpu


def _h_body(comp_ref, seq_ref, w_ref, h_ref):
    w = w_ref[...] * comp_ref[0, 0]
    h_ref[...] = jnp.dot(seq_ref[...], w, preferred_element_type=jnp.float32)


def _spmm_body(adj_ref, h_ref, out_ref):
    acc = jnp.dot(adj_ref[...], h_ref[...],
                  preferred_element_type=jnp.float32)
    out_ref[...] = jnp.maximum(acc, 0.0)


def kernel(seqs, adjs, comp, weight):
    seq = seqs[0]
    adj = adjs[0]
    w = weight[0]
    n, in_ft = seq.shape
    out_ft = w.shape[1]

    h = pl.pallas_call(
        _h_body,
        in_specs=[
            pl.BlockSpec(memory_space=pltpu.SMEM),
            pl.BlockSpec((n, in_ft), lambda: (0, 0)),
            pl.BlockSpec((in_ft, out_ft), lambda: (0, 0)),
        ],
        out_specs=pl.BlockSpec((n, out_ft), lambda: (0, 0)),
        out_shape=jax.ShapeDtypeStruct((n, out_ft), jnp.float32),
    )(comp, seq, w)

    bi = 200
    grid = (n // bi,)
    out = pl.pallas_call(
        _spmm_body,
        grid=grid,
        in_specs=[
            pl.BlockSpec((bi, n), lambda i: (i, 0)),
            pl.BlockSpec((n, out_ft), lambda i: (0, 0)),
        ],
        out_specs=pl.BlockSpec((bi, out_ft), lambda i: (i, 0)),
        out_shape=jax.ShapeDtypeStruct((n, out_ft), jnp.float32),
        compiler_params=pltpu.CompilerParams(
            dimension_semantics=("arbitrary",)),
    )(adj, h)
    return out


# speedup vs baseline: 1.0368x; 1.0368x over previous
import jax
import jax.numpy as jnp
from jax.experimental import pallas as pl
from jax.experimental.pallas import tpu as pltpu


def _fused_body(comp_ref, adj_ref, seq_ref, w_ref, out_ref, h_scr):
    @pl.when(pl.program_id(0) == 0)
    def _make_h():
        w = w_ref[...] * comp_ref[0, 0]
        h_scr[...] = jnp.dot(seq_ref[...], w,
                             preferred_element_type=jnp.float32)

    acc = jnp.dot(adj_ref[...], h_scr[...],
                  preferred_element_type=jnp.float32)
    out_ref[...] = jnp.maximum(acc, 0.0)


def kernel(seqs, adjs, comp, weight):
    seq = seqs[0]
    adj = adjs[0]
    w = weight[0]
    n, in_ft = seq.shape
    out_ft = w.shape[1]

    bi = 200
    grid = (n // bi,)
    out = pl.pallas_call(
        _fused_body,
        grid=grid,
        in_specs=[
            pl.BlockSpec(memory_space=pltpu.SMEM),
            pl.BlockSpec((bi, n), lambda i: (i, 0)),
            pl.BlockSpec((n, in_ft), lambda i: (0, 0)),
            pl.BlockSpec((in_ft, out_ft), lambda i: (0, 0)),
        ],
        out_specs=pl.BlockSpec((bi, out_ft), lambda i: (i, 0)),
        out_shape=jax.ShapeDtypeStruct((n, out_ft), jnp.float32),
        scratch_shapes=[pltpu.VMEM((n, out_ft), jnp.float32)],
        compiler_params=pltpu.CompilerParams(
            dimension_semantics=("arbitrary",)),
    )(comp, adj, seq, w)
    return out


# speedup vs baseline: 1.0453x; 1.0083x over previous
import jax
import jax.numpy as jnp
from jax.experimental import pallas as pl
from jax.experimental.pallas import tpu as pltpu


def _fused_body(comp_ref, adj_ref, seq_ref, w_ref, out_ref, h_scr):
    @pl.when(pl.program_id(0) == 0)
    def _make_h():
        w = w_ref[...] * comp_ref[0, 0]
        h_scr[...] = jnp.dot(seq_ref[...], w,
                             preferred_element_type=jnp.float32)

    acc = jnp.dot(adj_ref[...].astype(jnp.bfloat16),
                  h_scr[...].astype(jnp.bfloat16),
                  preferred_element_type=jnp.float32)
    out_ref[...] = jnp.maximum(acc, 0.0)


def kernel(seqs, adjs, comp, weight):
    seq = seqs[0]
    adj = adjs[0]
    w = weight[0]
    n, in_ft = seq.shape
    out_ft = w.shape[1]

    bi = 200
    grid = (n // bi,)
    out = pl.pallas_call(
        _fused_body,
        grid=grid,
        in_specs=[
            pl.BlockSpec(memory_space=pltpu.SMEM),
            pl.BlockSpec((bi, n), lambda i: (i, 0)),
            pl.BlockSpec((n, in_ft), lambda i: (0, 0)),
            pl.BlockSpec((in_ft, out_ft), lambda i: (0, 0)),
        ],
        out_specs=pl.BlockSpec((bi, out_ft), lambda i: (i, 0)),
        out_shape=jax.ShapeDtypeStruct((n, out_ft), jnp.float32),
        scratch_shapes=[pltpu.VMEM((n, out_ft), jnp.float32)],
        compiler_params=pltpu.CompilerParams(
            dimension_semantics=("arbitrary",)),
    )(comp, adj, seq, w)
    return out
